# baseline (speedup 1.0000x reference)
import functools

import jax
import jax.numpy as jnp
from jax import lax
from jax.experimental import pallas as pl
from jax.experimental.pallas import tpu as pltpu
from jax.experimental.pallas import tpu_sc as plsc

NC = 2
NS = 16
L = 16
NW = NC * NS
B = 16384
F = 64
BPW = B // NW
CHUNK = 128
NCHUNK = BPW // CHUNK

_mesh = plsc.VectorSubcoreMesh(core_axis_name="c", subcore_axis_name="s")


@functools.partial(
    pl.kernel,
    out_type=jax.ShapeDtypeStruct((B,), jnp.float32),
    mesh=_mesh,
    compiler_params=pltpu.CompilerParams(
        needs_layout_passes=False, use_tc_tiling_on_sc=False),
    scratch_types=[
        pltpu.VMEM((NCHUNK, CHUNK), jnp.int32),
        pltpu.VMEM((NCHUNK, CHUNK), jnp.int32),
        pltpu.VMEM((BPW, F), jnp.float32),
        pltpu.VMEM((BPW, F), jnp.float32),
        pltpu.VMEM((F, L), jnp.float32),
        pltpu.VMEM((L,), jnp.float32),
        pltpu.VMEM((BPW,), jnp.float32),
        pltpu.SemaphoreType.DMA,
    ],
)
def _gmf_sc(user_hbm, item_hbm, ut_hbm, it_hbm, wb_hbm, bb_hbm, out_hbm,
            uidx_v, iidx_v, urows_v, irows_v, w_v, b_v, out_v, sem):
    wid = lax.axis_index("s") * NC + lax.axis_index("c")
    base = wid * BPW

    pltpu.sync_copy(user_hbm.at[pl.ds(wid * NCHUNK, NCHUNK)], uidx_v)
    pltpu.sync_copy(item_hbm.at[pl.ds(wid * NCHUNK, NCHUNK)], iidx_v)

    copies = []
    for c in range(NCHUNK):
        rows = pl.ds(c * CHUNK, CHUNK)
        copies.append(pltpu.async_copy(ut_hbm.at[uidx_v.at[c]], urows_v.at[rows], sem))
        copies.append(pltpu.async_copy(it_hbm.at[iidx_v.at[c]], irows_v.at[rows], sem))
    pltpu.sync_copy(wb_hbm, w_v)
    pltpu.sync_copy(bb_hbm, b_v)
    for cp in copies:
        cp.wait()

    bias = b_v[...]

    def group(g, carry):
        row = g * L + lax.iota(jnp.int32, L)
        acc = bias
        for f in range(F):
            col = jnp.full((L,), f, jnp.int32)
            uf = plsc.load_gather(urows_v, [row, col])
            vf = plsc.load_gather(irows_v, [row, col])
            acc = acc + uf * vf * w_v[f, :]
        out_v[pl.ds(g * L, L)] = 1.0 / (1.0 + jnp.exp(-acc))
        return carry

    lax.fori_loop(0, BPW // L, group, 0)

    pltpu.sync_copy(out_v, out_hbm.at[pl.ds(base, BPW)])


def kernel(user, item, user_table, item_table, W, b):
    wb = jnp.broadcast_to(W.reshape(F, 1), (F, L))
    bb = jnp.broadcast_to(b.reshape(1), (L,))
    return _gmf_sc(user.reshape(B // CHUNK, CHUNK), item.reshape(B // CHUNK, CHUNK),
                   user_table, item_table, wb, bb)
